# Initial kernel scaffold; baseline (speedup 1.0000x reference)
import jax
import jax.numpy as jnp
from jax.experimental import pallas as pl


def kernel(x, token_table, position_table):
    raise NotImplementedError("write your pallas kernel here")


# speedup vs baseline: 1.8141x; 1.8141x over previous
import functools

import jax
import jax.numpy as jnp
from jax import lax
from jax.experimental import pallas as pl
from jax.experimental.pallas import tpu as pltpu
from jax.experimental.pallas import tpu_sc as plsc

NC = 2
NS = 16
NW = NC * NS
L = 16

BB = 128


@functools.partial(jax.jit, static_argnums=(3, 4, 5))
def _emb_call(xT, token_table, posT, B, S, D):
    mesh = plsc.VectorSubcoreMesh(core_axis_name="c", subcore_axis_name="s")
    n_bblk = B // BB
    assert n_bblk == NW

    @functools.partial(
        pl.kernel,
        out_type=jax.ShapeDtypeStruct((S, D // 8, n_bblk, 8 * BB), jnp.float32),
        mesh=mesh,
        scratch_types=[
            pltpu.VMEM((S, D), jnp.float32),
            pltpu.VMEM((S, BB), jnp.int32),
            pltpu.VMEM((2, BB, D), jnp.float32),
            pltpu.VMEM((2, D // 8, 8 * BB), jnp.float32),
            pltpu.SemaphoreType.DMA,
            pltpu.SemaphoreType.DMA,
            pltpu.SemaphoreType.DMA,
            pltpu.SemaphoreType.DMA,
        ],
        compiler_params=pltpu.CompilerParams(
            use_tc_tiling_on_sc=False, needs_layout_passes=False),
    )
    def emb(xT_hbm, tok_hbm, pos_hbm, out_hbm, pos_v, idx_v, g_v, t_v,
            sem_g0, sem_g1, sem_o0, sem_o1):
        sem_g = (sem_g0, sem_g1)
        sem_o = (sem_o0, sem_o1)
        wid = lax.axis_index("c") * NS + lax.axis_index("s")

        pltpu.sync_copy(pos_hbm, pos_v)
        pltpu.sync_copy(xT_hbm.at[:, pl.ds(wid * BB, BB)], idx_v)

        def start_gather(s, buf):
            pltpu.async_copy(tok_hbm.at[idx_v.at[s]], g_v.at[buf], sem_g[buf])

        def wait_gather(s, buf):
            pltpu.make_async_copy(
                tok_hbm.at[idx_v.at[s]], g_v.at[buf], sem_g[buf]).wait()

        def start_out(s, buf):
            pltpu.async_copy(t_v.at[buf], out_hbm.at[s, :, wid], sem_o[buf])

        def wait_out(s, buf):
            pltpu.make_async_copy(
                t_v.at[buf], out_hbm.at[s, :, wid], sem_o[buf]).wait()

        lane = jnp.arange(L, dtype=jnp.int32)
        rowids = [lane + L * bb for bb in range(BB // L)]
        bufid = [jnp.zeros((L,), jnp.int32), jnp.ones((L,), jnp.int32)]
        laneful = [jnp.full((L,), i, dtype=jnp.int32) for i in range(L)]
        inb = lax.GatherScatterMode.PROMISE_IN_BOUNDS

        def transpose_add(s, buf):
            def body(dg, carry):
                pvec = pos_v[s, pl.ds(dg * L, L)]
                dbase = dg * L
                for dl in range(L):
                    d = dbase + dl
                    p = pvec.at[laneful[dl]].get(mode=inb)
                    colid = jnp.full((L,), d, dtype=jnp.int32)
                    dt = d // 8
                    di = dl % 8
                    for bb in range(BB // L):
                        v = plsc.load_gather(
                            g_v, [bufid[buf], rowids[bb], colid])
                        t_v[buf, dt, pl.ds(di * BB + bb * L, L)] = v + p
                return carry
            lax.fori_loop(0, D // L, body, 0)

        def chunk_step(s, buf, k):
            nbuf = 1 - buf

            wait_gather(s, buf)

            @pl.when(s + 1 < S)
            def _():
                start_gather(s + 1, nbuf)

            @pl.when(k > 0)
            def _():
                wait_out(s - 2, buf)

            transpose_add(s, buf)
            start_out(s, buf)

        start_gather(0, 0)

        def loop_body(k, carry):
            chunk_step(2 * k, 0, k)
            chunk_step(2 * k + 1, 1, k)
            return carry

        lax.fori_loop(0, S // 2, loop_body, 0)
        wait_out(S - 2, 0)
        wait_out(S - 1, 1)

    return emb(xT, token_table, posT)


def kernel(x, token_table, position_table):
    B, S = x.shape
    D = token_table.shape[1]
    xT = x.T.astype(jnp.int32)
    pos = position_table[:S]
    out5 = _emb_call(xT, token_table, pos, B, S, D)
    out = out5.reshape(S, D // 8, B // BB, 8, BB)
    out = out.transpose(2, 4, 0, 1, 3).reshape(B, S, D)
    return out


# speedup vs baseline: 3.1159x; 1.7176x over previous
import functools

import jax
import jax.numpy as jnp
from jax import lax
from jax.experimental import pallas as pl
from jax.experimental.pallas import tpu as pltpu
from jax.experimental.pallas import tpu_sc as plsc

NC = 2
NS = 16
NW = NC * NS
L = 16

BB = 128
TW = BB + 1


@functools.partial(jax.jit, static_argnums=(3, 4, 5))
def _emb_call(xT, token_table, posT, B, S, D):
    mesh = plsc.VectorSubcoreMesh(core_axis_name="c", subcore_axis_name="s")
    n_bblk = B // BB
    assert n_bblk == NW

    @functools.partial(
        pl.kernel,
        out_type=jax.ShapeDtypeStruct((S, D // 8, n_bblk, 8, BB), jnp.float32),
        mesh=mesh,
        scratch_types=[
            pltpu.VMEM((S, D), jnp.float32),
            pltpu.VMEM((S, BB), jnp.int32),
            pltpu.VMEM((2, BB, D), jnp.float32),
            pltpu.VMEM((2, D, TW), jnp.float32),
            pltpu.SemaphoreType.DMA,
            pltpu.SemaphoreType.DMA,
            pltpu.SemaphoreType.DMA,
            pltpu.SemaphoreType.DMA,
        ],
        compiler_params=pltpu.CompilerParams(
            use_tc_tiling_on_sc=False, needs_layout_passes=False),
    )
    def emb(xT_hbm, tok_hbm, pos_hbm, out_hbm, pos_v, idx_v, g_v, t_v,
            sem_g0, sem_g1, sem_o0, sem_o1):
        sem_g = (sem_g0, sem_g1)
        sem_o = (sem_o0, sem_o1)
        wid = lax.axis_index("c") * NS + lax.axis_index("s")

        pltpu.sync_copy(pos_hbm, pos_v)
        pltpu.sync_copy(xT_hbm.at[:, pl.ds(wid * BB, BB)], idx_v)

        def start_gather(s, buf):
            pltpu.async_copy(tok_hbm.at[idx_v.at[s]], g_v.at[buf], sem_g[buf])

        def wait_gather(s, buf):
            pltpu.make_async_copy(
                tok_hbm.at[idx_v.at[s]], g_v.at[buf], sem_g[buf]).wait()

        def start_out(s, buf):
            for dt in range(D // 8):
                pltpu.async_copy(
                    t_v.at[buf, pl.ds(8 * dt, 8), pl.ds(0, BB)],
                    out_hbm.at[s, dt, wid], sem_o[buf])

        def wait_out(s, buf):
            for dt in range(D // 8):
                pltpu.make_async_copy(
                    t_v.at[buf, pl.ds(8 * dt, 8), pl.ds(0, BB)],
                    out_hbm.at[s, dt, wid], sem_o[buf]).wait()

        lane = jnp.arange(L, dtype=jnp.int32)
        dvecs = [lane + L * i for i in range(D // L)]
        bufid = [jnp.zeros((L,), jnp.int32), jnp.ones((L,), jnp.int32)]

        def transpose_add(s, buf):
            pv = [pos_v[s, pl.ds(i * L, L)] for i in range(D // L)]

            def body(bi, carry):
                bisplat = jnp.broadcast_to(bi, (L,)).astype(jnp.int32)
                for i in range(D // L):
                    v = g_v[buf, bi, pl.ds(i * L, L)] + pv[i]
                    plsc.store_scatter(
                        t_v, [bufid[buf], dvecs[i], bisplat], v)
                return carry
            lax.fori_loop(0, BB, body, 0)

        def chunk_step(s, buf, k):
            nbuf = 1 - buf

            wait_gather(s, buf)

            @pl.when(s + 1 < S)
            def _():
                start_gather(s + 1, nbuf)

            @pl.when(k > 0)
            def _():
                wait_out(s - 2, buf)

            transpose_add(s, buf)
            start_out(s, buf)

        start_gather(0, 0)

        def loop_body(k, carry):
            chunk_step(2 * k, 0, k)
            chunk_step(2 * k + 1, 1, k)
            return carry

        lax.fori_loop(0, S // 2, loop_body, 0)
        wait_out(S - 2, 0)
        wait_out(S - 1, 1)

    return emb(xT, token_table, posT)


def kernel(x, token_table, position_table):
    B, S = x.shape
    D = token_table.shape[1]
    xT = x.T.astype(jnp.int32)
    pos = position_table[:S]
    out5 = _emb_call(xT, token_table, pos, B, S, D)
    out = out5.transpose(2, 4, 0, 1, 3).reshape(B, S, D)
    return out


# speedup vs baseline: 3.1688x; 1.0170x over previous
import functools

import jax
import jax.numpy as jnp
from jax import lax
from jax.experimental import pallas as pl
from jax.experimental.pallas import tpu as pltpu
from jax.experimental.pallas import tpu_sc as plsc

NC = 2
NS = 16
NW = NC * NS
L = 16

BB = 128
TW = BB + 1
DEPTH = 4


@functools.partial(jax.jit, static_argnums=(3, 4, 5))
def _emb_call(xT, token_table, pos, B, S, D):
    mesh = plsc.VectorSubcoreMesh(core_axis_name="c", subcore_axis_name="s")
    n_bblk = B // BB
    assert n_bblk == NW and S % DEPTH == 0

    @functools.partial(
        pl.kernel,
        out_type=jax.ShapeDtypeStruct((S, D // 8, n_bblk, 8, BB), jnp.float32),
        mesh=mesh,
        scratch_types=[
            pltpu.VMEM((S, D), jnp.float32),
            pltpu.VMEM((S, BB), jnp.int32),
            pltpu.VMEM((DEPTH, BB, D), jnp.float32),
            pltpu.VMEM((DEPTH, D // 8, 8, TW), jnp.float32),
            pltpu.SemaphoreType.DMA((DEPTH,)),
            pltpu.SemaphoreType.DMA((DEPTH,)),
        ],
        compiler_params=pltpu.CompilerParams(
            use_tc_tiling_on_sc=False, needs_layout_passes=False),
    )
    def emb(xT_hbm, tok_hbm, pos_hbm, out_hbm, pos_v, idx_v, g_v, t_v,
            sem_g, sem_o):
        wid = lax.axis_index("c") * NS + lax.axis_index("s")

        pltpu.sync_copy(pos_hbm, pos_v)
        pltpu.sync_copy(xT_hbm.at[:, pl.ds(wid * BB, BB)], idx_v)

        def start_gather(s, buf):
            pltpu.async_copy(tok_hbm.at[idx_v.at[s]], g_v.at[buf],
                             sem_g.at[buf])

        def wait_gather(s, buf):
            pltpu.make_async_copy(tok_hbm.at[idx_v.at[s]], g_v.at[buf],
                                  sem_g.at[buf]).wait()

        def start_out(s, buf):
            pltpu.async_copy(t_v.at[buf, :, :, pl.ds(0, BB)],
                             out_hbm.at[s, :, wid], sem_o.at[buf])

        def wait_out(s, buf):
            pltpu.make_async_copy(t_v.at[buf, :, :, pl.ds(0, BB)],
                                  out_hbm.at[s, :, wid], sem_o.at[buf]).wait()

        lane = jnp.arange(L, dtype=jnp.int32)
        dtvecs = [(lane + L * i) // 8 for i in range(D // L)]
        divecs = [(lane + L * i) % 8 for i in range(D // L)]
        bufids = [jnp.full((L,), b, dtype=jnp.int32) for b in range(DEPTH)]

        def transpose_add(s, buf):
            pv = [pos_v[s, pl.ds(i * L, L)] for i in range(D // L)]

            def body(b2, carry):
                for r in range(2):
                    bi = b2 * 2 + r
                    bisplat = jnp.broadcast_to(bi, (L,)).astype(jnp.int32)
                    for i in range(D // L):
                        v = g_v[buf, bi, pl.ds(i * L, L)] + pv[i]
                        plsc.store_scatter(
                            t_v, [bufids[buf], dtvecs[i], divecs[i], bisplat],
                            v)
                return carry
            lax.fori_loop(0, BB // 2, body, 0)

        def chunk_step(s, buf, k):
            wait_gather(s, buf)

            @pl.when(s + 2 < S)
            def _():
                start_gather(s + 2, (buf + 2) % DEPTH)

            @pl.when(k > 0)
            def _():
                wait_out(s - DEPTH, buf)

            transpose_add(s, buf)
            start_out(s, buf)

        start_gather(0, 0)
        start_gather(1, 1)

        def loop_body(k, carry):
            for j in range(DEPTH):
                chunk_step(DEPTH * k + j, j, k)
            return carry

        lax.fori_loop(0, S // DEPTH, loop_body, 0)
        for j in range(DEPTH):
            wait_out(S - DEPTH + j, j)

    return emb(xT, token_table, pos)


def kernel(x, token_table, position_table):
    B, S = x.shape
    D = token_table.shape[1]
    xT = x.T.astype(jnp.int32)
    pos = position_table[:S]
    out5 = _emb_call(xT, token_table, pos, B, S, D)
    out = out5.transpose(2, 4, 0, 1, 3).reshape(B, S, D)
    return out


# speedup vs baseline: 4.5656x; 1.4408x over previous
import functools

import jax
import jax.numpy as jnp
from jax import lax
from jax.experimental import pallas as pl
from jax.experimental.pallas import tpu as pltpu
from jax.experimental.pallas import tpu_sc as plsc

NC = 2
NS = 16
NW = NC * NS
L = 16

BB = 128
TW = BB + 1
DEPTH = 4


@functools.partial(jax.jit, static_argnums=(3, 4, 5))
def _emb_call(xT, token_table, pos, B, S, D):
    mesh = plsc.VectorSubcoreMesh(core_axis_name="c", subcore_axis_name="s")
    n_bblk = B // BB
    assert n_bblk == NW and S % DEPTH == 0

    @functools.partial(
        pl.kernel,
        out_type=jax.ShapeDtypeStruct((S, D // 8, n_bblk, 8, BB), jnp.float32),
        mesh=mesh,
        scratch_types=[
            pltpu.VMEM((S, D), jnp.float32),
            pltpu.VMEM((S, BB), jnp.int32),
            pltpu.VMEM((DEPTH, BB, D), jnp.float32),
            pltpu.VMEM((DEPTH, D // 8, 8, TW), jnp.float32),
            pltpu.SemaphoreType.DMA((DEPTH,)),
            pltpu.SemaphoreType.DMA((DEPTH,)),
        ],
        compiler_params=pltpu.CompilerParams(
            use_tc_tiling_on_sc=False, needs_layout_passes=False),
    )
    def emb(xT_hbm, tok_hbm, pos_hbm, out_hbm, pos_v, idx_v, g_v, t_v,
            sem_g, sem_o):
        wid = lax.axis_index("c") * NS + lax.axis_index("s")

        pltpu.sync_copy(pos_hbm, pos_v)
        pltpu.sync_copy(xT_hbm.at[:, pl.ds(wid * BB, BB)], idx_v)

        def start_gather(s, buf):
            pltpu.async_copy(tok_hbm.at[idx_v.at[s]], g_v.at[buf],
                             sem_g.at[buf])

        def wait_gather(s, buf):
            pltpu.make_async_copy(tok_hbm.at[idx_v.at[s]], g_v.at[buf],
                                  sem_g.at[buf]).wait()

        def start_out(s, buf):
            pltpu.async_copy(t_v.at[buf, :, :, pl.ds(0, BB)],
                             out_hbm.at[s, :, wid], sem_o.at[buf])

        def wait_out(s, buf):
            pltpu.make_async_copy(t_v.at[buf, :, :, pl.ds(0, BB)],
                                  out_hbm.at[s, :, wid], sem_o.at[buf]).wait()

        lane = jnp.arange(L, dtype=jnp.int32)
        dtvecs = [(lane + L * i) // 8 for i in range(D // L)]
        divecs = [(lane + L * i) % 8 for i in range(D // L)]
        bufids = [jnp.full((L,), b, dtype=jnp.int32) for b in range(DEPTH)]

        def transpose_add(s, buf):
            pv = [pos_v[s, pl.ds(i * L, L)] for i in range(D // L)]

            @plsc.parallel_loop(0, BB, unroll=8)
            def body(bi):
                bisplat = jnp.broadcast_to(bi, (L,)).astype(jnp.int32)
                for i in range(D // L):
                    v = g_v[buf, bi, pl.ds(i * L, L)] + pv[i]
                    plsc.store_scatter(
                        t_v, [bufids[buf], dtvecs[i], divecs[i], bisplat], v)

        def chunk_step(s, buf, k):
            wait_gather(s, buf)

            @pl.when(s + 2 < S)
            def _():
                start_gather(s + 2, (buf + 2) % DEPTH)

            @pl.when(k > 0)
            def _():
                wait_out(s - DEPTH, buf)

            transpose_add(s, buf)
            start_out(s, buf)

        start_gather(0, 0)
        start_gather(1, 1)

        def loop_body(k, carry):
            for j in range(DEPTH):
                chunk_step(DEPTH * k + j, j, k)
            return carry

        lax.fori_loop(0, S // DEPTH, loop_body, 0)
        for j in range(DEPTH):
            wait_out(S - DEPTH + j, j)

    return emb(xT, token_table, pos)


def kernel(x, token_table, position_table):
    B, S = x.shape
    D = token_table.shape[1]
    xT = x.T.astype(jnp.int32)
    pos = position_table[:S]
    out5 = _emb_call(xT, token_table, pos, B, S, D)
    out = out5.transpose(2, 4, 0, 1, 3).reshape(B, S, D)
    return out
